# Initial kernel scaffold; baseline (speedup 1.0000x reference)
import jax
import jax.numpy as jnp
from jax.experimental import pallas as pl


def kernel(centor_word, window_word, neg_word, in_emb, out_emb):
    raise NotImplementedError("write your pallas kernel here")


# speedup vs baseline: 3.0395x; 3.0395x over previous
import functools

import jax
import jax.numpy as jnp
from jax import lax
from jax.experimental import pallas as pl
from jax.experimental.pallas import tpu as pltpu
from jax.experimental.pallas import tpu_sc as plsc

_NC = 2
_NS = 16
_NW = _NC * _NS


def _sc_gather(in_emb, out_emb, c_idx, wn_idx):
    vocab, emb = in_emb.shape
    (b,) = c_idx.shape
    (wn_tot,) = wn_idx.shape
    c_per_w = b // _NW
    wn_per_w = wn_tot // _NW
    chunk = 640
    n_chunks = wn_per_w // chunk
    assert wn_per_w % chunk == 0 and c_per_w % 8 == 0

    mesh = plsc.VectorSubcoreMesh(core_axis_name="c", subcore_axis_name="s")

    @functools.partial(
        pl.kernel,
        mesh=mesh,
        out_type=[
            jax.ShapeDtypeStruct((b, emb), jnp.float32),
            jax.ShapeDtypeStruct((wn_tot, emb), jnp.float32),
        ],
        scratch_types=[
            pltpu.VMEM((chunk,), jnp.int32),
            pltpu.VMEM((chunk, emb), jnp.float32),
            pltpu.SemaphoreType.DMA,
        ],
        compiler_params=pltpu.CompilerParams(use_tc_tiling_on_sc=False),
    )
    def k(in_emb_hbm, out_emb_hbm, c_idx_hbm, wn_idx_hbm, c_out, wn_out,
          idx_v, rows_v, sem):
        wid = lax.axis_index("s") * _NC + lax.axis_index("c")

        cbase = wid * c_per_w
        pltpu.sync_copy(c_idx_hbm.at[pl.ds(cbase, c_per_w)],
                        idx_v.at[pl.ds(0, c_per_w)])
        pltpu.async_copy(in_emb_hbm.at[idx_v.at[pl.ds(0, c_per_w)]],
                         rows_v.at[pl.ds(0, c_per_w)], sem).wait()
        pltpu.sync_copy(rows_v.at[pl.ds(0, c_per_w)],
                        c_out.at[pl.ds(cbase, c_per_w)])

        wbase = wid * wn_per_w

        def body(kk, _):
            off = wbase + kk * chunk
            pltpu.sync_copy(wn_idx_hbm.at[pl.ds(off, chunk)], idx_v)
            pltpu.async_copy(out_emb_hbm.at[idx_v], rows_v, sem).wait()
            pltpu.sync_copy(rows_v, wn_out.at[pl.ds(off, chunk)])
            return 0

        lax.fori_loop(0, n_chunks, body, 0)

    return k(in_emb, out_emb, c_idx, wn_idx)


def _log_sigmoid(x):
    return jnp.minimum(x, 0.0) - jnp.log1p(jnp.exp(-jnp.abs(x)))


def _tc_stats(center_rows, wn_rows3, w, n):
    b, emb = center_rows.shape
    wn = w + n
    blk = 256
    grid = b // blk

    def body(c_ref, x_ref, lp_ref, ln_ref):
        c = c_ref[...]
        x = x_ref[...]
        dots = jnp.sum(x * c[:, None, :], axis=-1)
        col = lax.broadcasted_iota(jnp.int32, (blk, wn), 1)
        is_w = col < w
        sgn = jnp.where(is_w, 1.0, -1.0)
        ls = _log_sigmoid(dots * sgn)
        lp = jnp.sum(jnp.where(is_w, ls, 0.0), axis=1)
        ln = jnp.sum(jnp.where(is_w, 0.0, ls), axis=1)
        lp_ref[...] = lp
        ln_ref[...] = ln

    lp2, ln2 = pl.pallas_call(
        body,
        grid=(grid,),
        in_specs=[
            pl.BlockSpec((blk, emb), lambda i: (i, 0)),
            pl.BlockSpec((blk, wn, emb), lambda i: (i, 0, 0)),
        ],
        out_specs=[
            pl.BlockSpec((blk,), lambda i: (i,)),
            pl.BlockSpec((blk,), lambda i: (i,)),
        ],
        out_shape=[
            jax.ShapeDtypeStruct((b,), jnp.float32),
            jax.ShapeDtypeStruct((b,), jnp.float32),
        ],
    )(center_rows, wn_rows3)
    return lp2, ln2


def _tc_broadcast(lp, ln):
    (b,) = lp.shape
    blk = 256
    grid = b // blk

    def body(lp_ref, ln_ref, o_ref):
        o_ref[...] = -(lp_ref[...] + ln_ref[...])

    return pl.pallas_call(
        body,
        grid=(grid,),
        in_specs=[
            pl.BlockSpec((1, b), lambda i: (0, 0)),
            pl.BlockSpec((blk, 1), lambda i: (i, 0)),
        ],
        out_specs=pl.BlockSpec((blk, b), lambda i: (i, 0)),
        out_shape=jax.ShapeDtypeStruct((b, b), jnp.float32),
    )(lp.reshape(1, b), ln.reshape(b, 1))


def kernel(centor_word, window_word, neg_word, in_emb, out_emb):
    b, w = window_word.shape
    _, n = neg_word.shape
    emb = in_emb.shape[1]

    c_idx = centor_word.astype(jnp.int32)
    wn_idx = jnp.concatenate(
        [window_word.astype(jnp.int32), neg_word.astype(jnp.int32)], axis=1
    ).reshape(b * (w + n))

    center_rows, wn_rows = _sc_gather(in_emb, out_emb, c_idx, wn_idx)
    wn_rows3 = wn_rows.reshape(b, w + n, emb)
    lp, ln = _tc_stats(center_rows, wn_rows3, w, n)
    return _tc_broadcast(lp, ln)


# speedup vs baseline: 3.6124x; 1.1885x over previous
import functools

import jax
import jax.numpy as jnp
from jax import lax
from jax.experimental import pallas as pl
from jax.experimental.pallas import tpu as pltpu
from jax.experimental.pallas import tpu_sc as plsc

_NC = 2
_NS = 16
_NW = _NC * _NS


def _sc_gather(in_emb, out_emb, c_idx_eo, wn_idx_eo):
    vocab, emb = in_emb.shape
    ch = c_idx_eo.shape[1]
    wh = wn_idx_eo.shape[1]
    c_per_w = ch // _NW
    wn_per_w = wh // _NW
    chunk = 320
    n_chunks = wn_per_w // chunk
    assert wn_per_w % chunk == 0 and c_per_w % 8 == 0 and emb == 64

    mesh = plsc.VectorSubcoreMesh(core_axis_name="c", subcore_axis_name="s")

    @functools.partial(
        pl.kernel,
        mesh=mesh,
        out_type=[
            jax.ShapeDtypeStruct((ch, 128), jnp.float32),
            jax.ShapeDtypeStruct((wh, 128), jnp.float32),
        ],
        scratch_types=[
            pltpu.VMEM((c_per_w,), jnp.int32),
            pltpu.VMEM((chunk,), jnp.int32),
            pltpu.VMEM((c_per_w, 64), jnp.float32),
            pltpu.VMEM((chunk, 64), jnp.float32),
            pltpu.SemaphoreType.DMA,
        ],
        compiler_params=pltpu.CompilerParams(use_tc_tiling_on_sc=False),
    )
    def k(in_emb_hbm, out_emb_hbm, c_idx_hbm, wn_idx_hbm, c_out, wn_out,
          cidx_v, idx_v, crow_v, rows_v, sem):
        wid = lax.axis_index("s") * _NC + lax.axis_index("c")

        cbase = wid * c_per_w
        for par in range(2):
            pltpu.sync_copy(c_idx_hbm.at[par, pl.ds(cbase, c_per_w)], cidx_v)
            pltpu.async_copy(in_emb_hbm.at[cidx_v], crow_v, sem).wait()
            pltpu.sync_copy(crow_v,
                            c_out.at[pl.ds(cbase, c_per_w),
                                     pl.ds(64 * par, 64)])

        wbase = wid * wn_per_w

        def body(kk, _):
            off = wbase + kk * chunk
            for par in range(2):
                pltpu.sync_copy(wn_idx_hbm.at[par, pl.ds(off, chunk)], idx_v)
                pltpu.async_copy(out_emb_hbm.at[idx_v], rows_v, sem).wait()
                pltpu.sync_copy(rows_v,
                                wn_out.at[pl.ds(off, chunk),
                                          pl.ds(64 * par, 64)])
            return 0

        lax.fori_loop(0, n_chunks, body, 0)

    return k(in_emb, out_emb, c_idx_eo, wn_idx_eo)


def _log_sigmoid(x):
    return jnp.minimum(x, 0.0) - jnp.log1p(jnp.exp(-jnp.abs(x)))


def _tc_stats(c2, x3, w, n):
    bh = c2.shape[0]
    wn = w + n
    blk = 256
    grid = bh // blk

    def body(c_ref, x_ref, lp_ref, ln_ref, d0_ref, d1_ref):
        c = c_ref[...]
        x = x_ref[...]
        prod = x * c[None, :, :]
        d0_ref[...] = jnp.sum(prod[:, :, :64], axis=-1)
        d1_ref[...] = jnp.sum(prod[:, :, 64:], axis=-1)
        d0 = d0_ref[...]
        d1 = d1_ref[...]
        krow = lax.broadcasted_iota(jnp.int32, (wn, blk), 0)
        is_w = krow < w
        sgn = jnp.where(is_w, 1.0, -1.0)
        ls0 = _log_sigmoid(d0 * sgn)
        ls1 = _log_sigmoid(d1 * sgn)
        lp0 = jnp.sum(jnp.where(is_w, ls0, 0.0), axis=0)
        lp1 = jnp.sum(jnp.where(is_w, ls1, 0.0), axis=0)
        ln0 = jnp.sum(jnp.where(is_w, 0.0, ls0), axis=0)
        ln1 = jnp.sum(jnp.where(is_w, 0.0, ls1), axis=0)
        lp_ref[...] = jnp.stack([lp0, lp1], axis=-1)
        ln_ref[...] = jnp.stack([ln0, ln1], axis=-1)

    lp2, ln2 = pl.pallas_call(
        body,
        grid=(grid,),
        in_specs=[
            pl.BlockSpec((blk, 128), lambda i: (i, 0)),
            pl.BlockSpec((wn, blk, 128), lambda i: (0, i, 0)),
        ],
        out_specs=[
            pl.BlockSpec((blk, 2), lambda i: (i, 0)),
            pl.BlockSpec((blk, 2), lambda i: (i, 0)),
        ],
        out_shape=[
            jax.ShapeDtypeStruct((bh, 2), jnp.float32),
            jax.ShapeDtypeStruct((bh, 2), jnp.float32),
        ],
        scratch_shapes=[
            pltpu.VMEM((wn, blk), jnp.float32),
            pltpu.VMEM((wn, blk), jnp.float32),
        ],
    )(c2, x3)
    return lp2, ln2


def _tc_broadcast(lp, ln):
    (b,) = lp.shape
    blk = 256
    grid = b // blk

    def body(lp_ref, ln_ref, o_ref):
        o_ref[...] = -(lp_ref[...] + ln_ref[...])

    return pl.pallas_call(
        body,
        grid=(grid,),
        in_specs=[
            pl.BlockSpec((1, b), lambda i: (0, 0)),
            pl.BlockSpec((blk, 1), lambda i: (i, 0)),
        ],
        out_specs=pl.BlockSpec((blk, b), lambda i: (i, 0)),
        out_shape=jax.ShapeDtypeStruct((b, b), jnp.float32),
    )(lp.reshape(1, b), ln.reshape(b, 1))


def kernel(centor_word, window_word, neg_word, in_emb, out_emb):
    b, w = window_word.shape
    _, n = neg_word.shape
    emb = in_emb.shape[1]
    wn = w + n

    c_idx = centor_word.astype(jnp.int32)
    wn_idx = jnp.concatenate(
        [window_word.astype(jnp.int32).T.reshape(b * w),
         neg_word.astype(jnp.int32).T.reshape(b * n)], axis=0)
    c_idx_eo = c_idx.reshape(b // 2, 2).T
    wn_idx_eo = wn_idx.reshape(b * wn // 2, 2).T

    c2, wn_rows = _sc_gather(in_emb, out_emb, c_idx_eo, wn_idx_eo)
    x3 = wn_rows.reshape(wn, b // 2, 128)
    lp2, ln2 = _tc_stats(c2, x3, w, n)
    lp = lp2.reshape(b)
    ln = ln2.reshape(b)
    return _tc_broadcast(lp, ln)


# speedup vs baseline: 4.9355x; 1.3663x over previous
import functools

import jax
import jax.numpy as jnp
from jax import lax
from jax.experimental import pallas as pl
from jax.experimental.pallas import tpu as pltpu
from jax.experimental.pallas import tpu_sc as plsc

_NC = 2
_NS = 16
_NW = _NC * _NS
_RING = 4


def _sc_gather(in_emb, out_emb, centor, window, neg):
    vocab, emb = in_emb.shape
    b, w = window.shape
    n = neg.shape[1]
    wn = w + n
    bw = b // _NW
    half = bw // 2
    jobs = 2 * wn
    assert emb == 64 and b % (2 * _NW) == 0

    mesh = plsc.VectorSubcoreMesh(core_axis_name="c", subcore_axis_name="s")

    @functools.partial(
        pl.kernel,
        mesh=mesh,
        out_type=[
            jax.ShapeDtypeStruct((_NW, half, 128), jnp.float32),
            jax.ShapeDtypeStruct((_NW, wn, half, 128), jnp.float32),
        ],
        scratch_types=[
            pltpu.VMEM((bw, w), jnp.int32),
            pltpu.VMEM((bw, n), jnp.int32),
            pltpu.VMEM((bw,), jnp.int32),
            pltpu.VMEM((jobs * half,), jnp.int32),
            pltpu.VMEM((bw,), jnp.int32),
            pltpu.VMEM((_RING, half, 64), jnp.float32),
            pltpu.VMEM((2, half, 64), jnp.float32),
            pltpu.SemaphoreType.DMA((_RING,)),
            pltpu.SemaphoreType.DMA((_RING,)),
            pltpu.SemaphoreType.DMA((2,)),
        ],
        compiler_params=pltpu.CompilerParams(use_tc_tiling_on_sc=False, needs_layout_passes=False),
    )
    def k(in_hbm, out_hbm, cen_hbm, win_hbm, neg_hbm, c3, x4,
          win_v, neg_v, cen_v, idxa_v, cidx_v, rows_v, crow_v,
          gsem, wsem, csem):
        wid = lax.axis_index("s") * _NC + lax.axis_index("c")
        b0 = wid * bw

        pltpu.sync_copy(win_hbm.at[pl.ds(b0, bw), :], win_v)
        pltpu.sync_copy(neg_hbm.at[pl.ds(b0, bw), :], neg_v)
        pltpu.sync_copy(cen_hbm.at[pl.ds(b0, bw)], cen_v)

        lane = lax.broadcasted_iota(jnp.int32, (16,), 0)

        def rearr(g, _):
            kk = g // 2
            par = g - 2 * kk
            col = jnp.full((16,), kk, jnp.int32)
            coln = jnp.full((16,), kk - w, jnp.int32)
            for q in range(half // 16):
                rows = lane * 2 + (32 * q + par)
                @pl.when(kk < w)
                def _():
                    vals = plsc.load_gather(win_v, [rows, col])
                    idxa_v[pl.ds(g * half + 16 * q, 16)] = vals
                @pl.when(kk >= w)
                def _():
                    vals = plsc.load_gather(neg_v, [rows, coln])
                    idxa_v[pl.ds(g * half + 16 * q, 16)] = vals
            return 0

        lax.fori_loop(0, jobs, rearr, 0)

        for par in range(2):
            for q in range(half // 16):
                rows = lane * 2 + (32 * q + par)
                cidx_v[pl.ds(par * half + 16 * q, 16)] = plsc.load_gather(
                    cen_v, [rows])

        def issue(j):
            slot = lax.rem(j, _RING)
            pltpu.async_copy(
                out_hbm.at[idxa_v.at[pl.ds(j * half, half)]],
                rows_v.at[slot], gsem.at[slot])

        def gwait(j):
            slot = lax.rem(j, _RING)
            pltpu.make_async_copy(
                out_hbm.at[idxa_v.at[pl.ds(j * half, half)]],
                rows_v.at[slot], gsem.at[slot]).wait()

        def wissue(j):
            slot = lax.rem(j, _RING)
            kk = j // 2
            par = j - 2 * kk
            pltpu.async_copy(
                rows_v.at[slot],
                x4.at[wid, kk, :, pl.ds(64 * par, 64)], wsem.at[slot])

        def wwait(slot):
            pltpu.make_async_copy(
                rows_v.at[slot],
                x4.at[wid, 0, :, pl.ds(0, 64)], wsem.at[slot]).wait()

        issue(0)
        issue(1)

        def body(j, _):
            nx = j + 2
            @pl.when(nx < jobs)
            def _():
                @pl.when(nx >= _RING)
                def _():
                    wwait(lax.rem(nx, _RING))
                issue(nx)
            gwait(j)
            wissue(j)
            return 0

        lax.fori_loop(0, jobs, body, 0)
        for s in range(_RING):
            wwait(s)

        for par in range(2):
            pltpu.async_copy(
                in_hbm.at[cidx_v.at[pl.ds(par * half, half)]],
                crow_v.at[par], csem.at[par]).wait()
            pltpu.sync_copy(crow_v.at[par],
                            c3.at[wid, :, pl.ds(64 * par, 64)])

    return k(in_emb, out_emb, centor, window, neg)


def _log_sigmoid(x):
    return jnp.minimum(x, 0.0) - jnp.log1p(jnp.exp(-jnp.abs(x)))


def _tc_stats(c3, x4, w, n):
    nwk = c3.shape[0]
    half = c3.shape[1]
    wn = w + n

    def body(c_ref, x_ref, lp_ref, ln_ref, d0_ref, d1_ref):
        c = c_ref[0]
        x = x_ref[0]
        prod = x * c[None, :, :]
        d0_ref[...] = jnp.sum(prod[:, :, :64], axis=-1)
        d1_ref[...] = jnp.sum(prod[:, :, 64:], axis=-1)
        d0 = d0_ref[...]
        d1 = d1_ref[...]
        krow = lax.broadcasted_iota(jnp.int32, (wn, half), 0)
        is_w = krow < w
        sgn = jnp.where(is_w, 1.0, -1.0)
        ls0 = _log_sigmoid(d0 * sgn)
        ls1 = _log_sigmoid(d1 * sgn)
        lp0 = jnp.sum(jnp.where(is_w, ls0, 0.0), axis=0)
        lp1 = jnp.sum(jnp.where(is_w, ls1, 0.0), axis=0)
        ln0 = jnp.sum(jnp.where(is_w, 0.0, ls0), axis=0)
        ln1 = jnp.sum(jnp.where(is_w, 0.0, ls1), axis=0)
        lp_ref[...] = jnp.stack([lp0, lp1], axis=-1)[None]
        ln_ref[...] = jnp.stack([ln0, ln1], axis=-1)[None]

    lp3, ln3 = pl.pallas_call(
        body,
        grid=(nwk,),
        in_specs=[
            pl.BlockSpec((1, half, 128), lambda i: (i, 0, 0)),
            pl.BlockSpec((1, wn, half, 128), lambda i: (i, 0, 0, 0)),
        ],
        out_specs=[
            pl.BlockSpec((1, half, 2), lambda i: (i, 0, 0)),
            pl.BlockSpec((1, half, 2), lambda i: (i, 0, 0)),
        ],
        out_shape=[
            jax.ShapeDtypeStruct((nwk, half, 2), jnp.float32),
            jax.ShapeDtypeStruct((nwk, half, 2), jnp.float32),
        ],
        scratch_shapes=[
            pltpu.VMEM((wn, half), jnp.float32),
            pltpu.VMEM((wn, half), jnp.float32),
        ],
    )(c3, x4)
    return lp3, ln3


def _tc_broadcast(lp, ln):
    (b,) = lp.shape
    blk = 256
    grid = b // blk

    def body(lp_ref, ln_ref, o_ref):
        o_ref[...] = -(lp_ref[...] + ln_ref[...])

    return pl.pallas_call(
        body,
        grid=(grid,),
        in_specs=[
            pl.BlockSpec((1, b), lambda i: (0, 0)),
            pl.BlockSpec((blk, 1), lambda i: (i, 0)),
        ],
        out_specs=pl.BlockSpec((blk, b), lambda i: (i, 0)),
        out_shape=jax.ShapeDtypeStruct((b, b), jnp.float32),
    )(lp.reshape(1, b), ln.reshape(b, 1))


def kernel(centor_word, window_word, neg_word, in_emb, out_emb):
    b, w = window_word.shape
    n = neg_word.shape[1]

    c3, x4 = _sc_gather(in_emb, out_emb,
                        centor_word.astype(jnp.int32),
                        window_word.astype(jnp.int32),
                        neg_word.astype(jnp.int32))
    lp3, ln3 = _tc_stats(c3, x4, w, n)
    lp = lp3.reshape(b)
    ln = ln3.reshape(b)
    return _tc_broadcast(lp, ln)
